# baseline (speedup 1.0000x reference)
import functools

import jax
import jax.numpy as jnp
from jax import lax
from jax.experimental import pallas as pl
from jax.experimental.pallas import tpu as pltpu
from jax.experimental.pallas import tpu_sc as plsc

B, N, D = 4096, 128, 64
SC_B = 2176
NC, NS, L = 2, 16, 16
NW = NC * NS
PER_W = SC_B // NW
CH = 1
SLAB = N * D
CHW = CH * SLAB
ROWS = CHW // N
NCHUNK = PER_W // CH
TB = 32


def _sc_body(p_hbm, g_hbm, m_hbm, out_hbm,
             pb0, gb0, mb0, pb1, gb1, mb1, pb2, gb2, mb2, pb3, gb3, mb3,
             accbuf, sem0, sem1, sem2, sem3):
    c = lax.axis_index("c")
    s = lax.axis_index("s")
    wid = s * NC + c
    base = wid * (PER_W * SLAB)
    bufs = ((pb0, gb0, mb0, sem0), (pb1, gb1, mb1, sem1),
            (pb2, gb2, mb2, sem2), (pb3, gb3, mb3, sem3))

    def start(ci, slot):
        pb, gb, mb, sem = bufs[slot]
        off = base + ci * CHW
        pltpu.async_copy(p_hbm.at[pl.ds(off, CHW)], pb, sem)
        pltpu.async_copy(g_hbm.at[pl.ds(off, CHW)], gb, sem)
        pltpu.async_copy(m_hbm.at[pl.ds(off, CHW)], mb, sem)

    def drain(slot):
        pb, gb, mb, sem = bufs[slot]
        pltpu.make_async_copy(p_hbm.at[pl.ds(0, CHW)], pb, sem).wait()
        pltpu.make_async_copy(g_hbm.at[pl.ds(0, CHW)], gb, sem).wait()
        pltpu.make_async_copy(m_hbm.at[pl.ds(0, CHW)], mb, sem).wait()

    def compute_chunk(slot, acc):
        pb, gb, mb, _ = bufs[slot]

        def row_body(r, a):
            rb = r * N
            for k in range(N // L):
                pos = rb + k * L
                pv = pb[pl.ds(pos, L)]
                gv = gb[pl.ds(pos, L)]
                iv = mb[pl.ds(pos, L)]
                e = pv - gv
                t = e * jnp.abs(e)
                addr = rb + iv
                pg = plsc.load_gather(pb, [addr])
                gg = plsc.load_gather(gb, [addr])
                a = a + t * jnp.sign(pg - gg)
            return a

        return lax.fori_loop(0, ROWS, row_body, acc)

    for slot in range(4):
        start(slot, slot)

    def ring_body(cj, acc):
        for slot in range(4):
            ci = cj * 4 + slot
            drain(slot)
            acc = compute_chunk(slot, acc)

            @pl.when(ci + 4 < NCHUNK)
            def _():
                start(ci + 4, slot)
        return acc

    acc = lax.fori_loop(0, NCHUNK // 4, ring_body, jnp.zeros((L,), jnp.float32))
    accbuf[...] = acc
    pltpu.sync_copy(accbuf, out_hbm.at[wid])


_sc_call = functools.partial(
    pl.kernel,
    mesh=plsc.VectorSubcoreMesh(core_axis_name="c", subcore_axis_name="s"),
    out_type=jax.ShapeDtypeStruct((NW, L), jnp.float32),
    compiler_params=pltpu.CompilerParams(needs_layout_passes=False),
    scratch_types=[
        pltpu.VMEM((CHW,), jnp.float32),
        pltpu.VMEM((CHW,), jnp.float32),
        pltpu.VMEM((CHW,), jnp.int32),
        pltpu.VMEM((CHW,), jnp.float32),
        pltpu.VMEM((CHW,), jnp.float32),
        pltpu.VMEM((CHW,), jnp.int32),
        pltpu.VMEM((CHW,), jnp.float32),
        pltpu.VMEM((CHW,), jnp.float32),
        pltpu.VMEM((CHW,), jnp.int32),
        pltpu.VMEM((CHW,), jnp.float32),
        pltpu.VMEM((CHW,), jnp.float32),
        pltpu.VMEM((CHW,), jnp.int32),
        pltpu.VMEM((L,), jnp.float32),
        pltpu.SemaphoreType.DMA,
        pltpu.SemaphoreType.DMA,
        pltpu.SemaphoreType.DMA,
        pltpu.SemaphoreType.DMA,
    ],
)(_sc_body)


def _tc_body(p_ref, g_ref, m_ref, o_ref, acc_ref):
    i = pl.program_id(0)

    @pl.when(i == 0)
    def _():
        acc_ref[...] = jnp.zeros_like(acc_ref)

    p = p_ref[...].reshape(TB * D, N)
    g = g_ref[...].reshape(TB * D, N)
    m = m_ref[...].reshape(TB * D, N)
    e = p - g
    t = e * jnp.abs(e)
    s = jnp.sign(e)
    gs = jnp.take_along_axis(s, m, axis=-1)
    acc_ref[...] += jnp.sum(t * gs, axis=0, keepdims=True)

    @pl.when(i == pl.num_programs(0) - 1)
    def _():
        o_ref[...] = acc_ref[...]


_tc_call = pl.pallas_call(
    _tc_body,
    grid=((B - SC_B) // TB,),
    in_specs=[
        pl.BlockSpec((TB, D, N), lambda i: (SC_B // TB + i, 0, 0)),
        pl.BlockSpec((TB, D, N), lambda i: (SC_B // TB + i, 0, 0)),
        pl.BlockSpec((TB, D, N), lambda i: (SC_B // TB + i, 0, 0)),
    ],
    out_specs=pl.BlockSpec((1, N), lambda i: (0, 0)),
    out_shape=jax.ShapeDtypeStruct((1, N), jnp.float32),
    scratch_shapes=[pltpu.VMEM((1, N), jnp.float32)],
    compiler_params=pltpu.CompilerParams(
        dimension_semantics=("arbitrary",),
    ),
)


@jax.jit
def kernel(predictions, ground_truth, anchor_masks):
    p3 = jnp.transpose(predictions, (0, 2, 1))
    g3 = jnp.transpose(ground_truth, (0, 2, 1))
    m3 = jnp.transpose(anchor_masks.astype(jnp.int32), (0, 2, 1))
    sc_partials = _sc_call(p3.reshape(-1), g3.reshape(-1), m3.reshape(-1))
    tc_partials = _tc_call(p3, g3, m3)
    total = jnp.sum(sc_partials) + jnp.sum(tc_partials)
    return total / (B * N * D)


# speedup vs baseline: 1.0427x; 1.0427x over previous
import functools

import jax
import jax.numpy as jnp
from jax import lax
from jax.experimental import pallas as pl
from jax.experimental.pallas import tpu as pltpu
from jax.experimental.pallas import tpu_sc as plsc

B, N, D = 4096, 128, 64
SC_B = 1920
NC, NS, L = 2, 16, 16
NW = NC * NS
PER_W = SC_B // NW
CH = 1
SLAB = N * D
CHW = CH * SLAB
ROWS = CHW // N
NCHUNK = PER_W // CH
TB = 32


def _sc_body(p_hbm, g_hbm, m_hbm, out_hbm,
             pb0, gb0, mb0, pb1, gb1, mb1, pb2, gb2, mb2, pb3, gb3, mb3,
             accbuf, sem0, sem1, sem2, sem3):
    c = lax.axis_index("c")
    s = lax.axis_index("s")
    wid = s * NC + c
    base = wid * (PER_W * SLAB)
    bufs = ((pb0, gb0, mb0, sem0), (pb1, gb1, mb1, sem1),
            (pb2, gb2, mb2, sem2), (pb3, gb3, mb3, sem3))

    def start(ci, slot):
        pb, gb, mb, sem = bufs[slot]
        off = base + ci * CHW
        pltpu.async_copy(p_hbm.at[pl.ds(off, CHW)], pb, sem)
        pltpu.async_copy(g_hbm.at[pl.ds(off, CHW)], gb, sem)
        pltpu.async_copy(m_hbm.at[pl.ds(off, CHW)], mb, sem)

    def drain(slot):
        pb, gb, mb, sem = bufs[slot]
        pltpu.make_async_copy(p_hbm.at[pl.ds(0, CHW)], pb, sem).wait()
        pltpu.make_async_copy(g_hbm.at[pl.ds(0, CHW)], gb, sem).wait()
        pltpu.make_async_copy(m_hbm.at[pl.ds(0, CHW)], mb, sem).wait()

    def compute_chunk(slot, acc):
        pb, gb, mb, _ = bufs[slot]

        def row_body(r, a):
            rb = r * N
            for k in range(N // L):
                pos = rb + k * L
                pv = pb[pl.ds(pos, L)]
                gv = gb[pl.ds(pos, L)]
                iv = mb[pl.ds(pos, L)]
                e = pv - gv
                t = e * jnp.abs(e)
                addr = rb + iv
                pg = plsc.load_gather(pb, [addr])
                gg = plsc.load_gather(gb, [addr])
                a = a + t * jnp.sign(pg - gg)
            return a

        return lax.fori_loop(0, ROWS, row_body, acc)

    for slot in range(4):
        start(slot, slot)

    def ring_body(cj, acc):
        for slot in range(4):
            ci = cj * 4 + slot
            drain(slot)
            acc = compute_chunk(slot, acc)

            @pl.when(ci + 4 < NCHUNK)
            def _():
                start(ci + 4, slot)
        return acc

    acc = lax.fori_loop(0, NCHUNK // 4, ring_body, jnp.zeros((L,), jnp.float32))
    accbuf[...] = acc
    pltpu.sync_copy(accbuf, out_hbm.at[wid])


_sc_call = functools.partial(
    pl.kernel,
    mesh=plsc.VectorSubcoreMesh(core_axis_name="c", subcore_axis_name="s"),
    out_type=jax.ShapeDtypeStruct((NW, L), jnp.float32),
    compiler_params=pltpu.CompilerParams(needs_layout_passes=False),
    scratch_types=[
        pltpu.VMEM((CHW,), jnp.float32),
        pltpu.VMEM((CHW,), jnp.float32),
        pltpu.VMEM((CHW,), jnp.int32),
        pltpu.VMEM((CHW,), jnp.float32),
        pltpu.VMEM((CHW,), jnp.float32),
        pltpu.VMEM((CHW,), jnp.int32),
        pltpu.VMEM((CHW,), jnp.float32),
        pltpu.VMEM((CHW,), jnp.float32),
        pltpu.VMEM((CHW,), jnp.int32),
        pltpu.VMEM((CHW,), jnp.float32),
        pltpu.VMEM((CHW,), jnp.float32),
        pltpu.VMEM((CHW,), jnp.int32),
        pltpu.VMEM((L,), jnp.float32),
        pltpu.SemaphoreType.DMA,
        pltpu.SemaphoreType.DMA,
        pltpu.SemaphoreType.DMA,
        pltpu.SemaphoreType.DMA,
    ],
)(_sc_body)


def _tc_body(p_ref, g_ref, m_ref, o_ref, acc_ref):
    i = pl.program_id(0)

    @pl.when(i == 0)
    def _():
        acc_ref[...] = jnp.zeros_like(acc_ref)

    p = p_ref[...].reshape(TB * D, N)
    g = g_ref[...].reshape(TB * D, N)
    m = m_ref[...].reshape(TB * D, N)
    e = p - g
    t = e * jnp.abs(e)
    s = jnp.sign(e)
    gs = jnp.take_along_axis(s, m, axis=-1)
    acc_ref[...] += jnp.sum(t * gs, axis=0, keepdims=True)

    @pl.when(i == pl.num_programs(0) - 1)
    def _():
        o_ref[...] = acc_ref[...]


_tc_call = pl.pallas_call(
    _tc_body,
    grid=((B - SC_B) // TB,),
    in_specs=[
        pl.BlockSpec((TB, D, N), lambda i: (SC_B // TB + i, 0, 0)),
        pl.BlockSpec((TB, D, N), lambda i: (SC_B // TB + i, 0, 0)),
        pl.BlockSpec((TB, D, N), lambda i: (SC_B // TB + i, 0, 0)),
    ],
    out_specs=pl.BlockSpec((1, N), lambda i: (0, 0)),
    out_shape=jax.ShapeDtypeStruct((1, N), jnp.float32),
    scratch_shapes=[pltpu.VMEM((1, N), jnp.float32)],
    compiler_params=pltpu.CompilerParams(
        dimension_semantics=("arbitrary",),
    ),
)


@jax.jit
def kernel(predictions, ground_truth, anchor_masks):
    p3 = jnp.transpose(predictions, (0, 2, 1))
    g3 = jnp.transpose(ground_truth, (0, 2, 1))
    m3 = jnp.transpose(anchor_masks.astype(jnp.int32), (0, 2, 1))
    sc_partials = _sc_call(p3.reshape(-1), g3.reshape(-1), m3.reshape(-1))
    tc_partials = _tc_call(p3, g3, m3)
    total = jnp.sum(sc_partials) + jnp.sum(tc_partials)
    return total / (B * N * D)


# speedup vs baseline: 1.0735x; 1.0295x over previous
import functools

import jax
import jax.numpy as jnp
from jax import lax
from jax.experimental import pallas as pl
from jax.experimental.pallas import tpu as pltpu
from jax.experimental.pallas import tpu_sc as plsc

B, N, D = 4096, 128, 64
SC_B = 2048
NC, NS, L = 2, 16, 16
NW = NC * NS
PER_W = SC_B // NW
CH = 1
SLAB = N * D
CHW = CH * SLAB
ROWS = CHW // N
NCHUNK = PER_W // CH
TB = 32


def _sc_body(p_hbm, g_hbm, m_hbm, out_hbm,
             pb0, gb0, mb0, pb1, gb1, mb1, pb2, gb2, mb2, pb3, gb3, mb3,
             accbuf, sem0, sem1, sem2, sem3):
    c = lax.axis_index("c")
    s = lax.axis_index("s")
    wid = s * NC + c
    base = wid * (PER_W * SLAB)
    bufs = ((pb0, gb0, mb0, sem0), (pb1, gb1, mb1, sem1),
            (pb2, gb2, mb2, sem2), (pb3, gb3, mb3, sem3))

    def start(ci, slot):
        pb, gb, mb, sem = bufs[slot]
        off = base + ci * CHW
        pltpu.async_copy(p_hbm.at[pl.ds(off, CHW)], pb, sem)
        pltpu.async_copy(g_hbm.at[pl.ds(off, CHW)], gb, sem)
        pltpu.async_copy(m_hbm.at[pl.ds(off, CHW)], mb, sem)

    def drain(slot):
        pb, gb, mb, sem = bufs[slot]
        pltpu.make_async_copy(p_hbm.at[pl.ds(0, CHW)], pb, sem).wait()
        pltpu.make_async_copy(g_hbm.at[pl.ds(0, CHW)], gb, sem).wait()
        pltpu.make_async_copy(m_hbm.at[pl.ds(0, CHW)], mb, sem).wait()

    def compute_chunk(slot, acc):
        pb, gb, mb, _ = bufs[slot]

        def row_body(r, a):
            rb = r * N
            for k in range(N // L):
                pos = rb + k * L
                pv = pb[pl.ds(pos, L)]
                gv = gb[pl.ds(pos, L)]
                iv = mb[pl.ds(pos, L)]
                e = pv - gv
                t = e * jnp.abs(e)
                addr = rb + iv
                pg = plsc.load_gather(pb, [addr])
                gg = plsc.load_gather(gb, [addr])
                a = a + t * jnp.sign(pg - gg)
            return a

        return lax.fori_loop(0, ROWS, row_body, acc)

    for slot in range(4):
        start(slot, slot)

    def ring_body(cj, acc):
        for slot in range(4):
            ci = cj * 4 + slot
            drain(slot)
            acc = compute_chunk(slot, acc)

            @pl.when(ci + 4 < NCHUNK)
            def _():
                start(ci + 4, slot)
        return acc

    acc = lax.fori_loop(0, NCHUNK // 4, ring_body, jnp.zeros((L,), jnp.float32))
    accbuf[...] = acc
    pltpu.sync_copy(accbuf, out_hbm.at[wid])


_sc_call = functools.partial(
    pl.kernel,
    mesh=plsc.VectorSubcoreMesh(core_axis_name="c", subcore_axis_name="s"),
    out_type=jax.ShapeDtypeStruct((NW, L), jnp.float32),
    compiler_params=pltpu.CompilerParams(needs_layout_passes=False),
    scratch_types=[
        pltpu.VMEM((CHW,), jnp.float32),
        pltpu.VMEM((CHW,), jnp.float32),
        pltpu.VMEM((CHW,), jnp.int32),
        pltpu.VMEM((CHW,), jnp.float32),
        pltpu.VMEM((CHW,), jnp.float32),
        pltpu.VMEM((CHW,), jnp.int32),
        pltpu.VMEM((CHW,), jnp.float32),
        pltpu.VMEM((CHW,), jnp.float32),
        pltpu.VMEM((CHW,), jnp.int32),
        pltpu.VMEM((CHW,), jnp.float32),
        pltpu.VMEM((CHW,), jnp.float32),
        pltpu.VMEM((CHW,), jnp.int32),
        pltpu.VMEM((L,), jnp.float32),
        pltpu.SemaphoreType.DMA,
        pltpu.SemaphoreType.DMA,
        pltpu.SemaphoreType.DMA,
        pltpu.SemaphoreType.DMA,
    ],
)(_sc_body)


def _tc_body(p_ref, g_ref, m_ref, o_ref, acc_ref):
    i = pl.program_id(0)

    @pl.when(i == 0)
    def _():
        acc_ref[...] = jnp.zeros_like(acc_ref)

    p = p_ref[...].reshape(TB * D, N)
    g = g_ref[...].reshape(TB * D, N)
    m = m_ref[...].reshape(TB * D, N)
    e = p - g
    t = e * jnp.abs(e)
    s = jnp.sign(e)
    gs = jnp.take_along_axis(s, m, axis=-1)
    acc_ref[...] += jnp.sum(t * gs, axis=0, keepdims=True)

    @pl.when(i == pl.num_programs(0) - 1)
    def _():
        o_ref[...] = acc_ref[...]


_tc_call = pl.pallas_call(
    _tc_body,
    grid=((B - SC_B) // TB,),
    in_specs=[
        pl.BlockSpec((TB, D, N), lambda i: (SC_B // TB + i, 0, 0)),
        pl.BlockSpec((TB, D, N), lambda i: (SC_B // TB + i, 0, 0)),
        pl.BlockSpec((TB, D, N), lambda i: (SC_B // TB + i, 0, 0)),
    ],
    out_specs=pl.BlockSpec((1, N), lambda i: (0, 0)),
    out_shape=jax.ShapeDtypeStruct((1, N), jnp.float32),
    scratch_shapes=[pltpu.VMEM((1, N), jnp.float32)],
    compiler_params=pltpu.CompilerParams(
        dimension_semantics=("arbitrary",),
    ),
)


@jax.jit
def kernel(predictions, ground_truth, anchor_masks):
    p3 = jnp.transpose(predictions, (0, 2, 1))
    g3 = jnp.transpose(ground_truth, (0, 2, 1))
    m3 = jnp.transpose(anchor_masks.astype(jnp.int32), (0, 2, 1))
    sc_partials = _sc_call(p3.reshape(-1), g3.reshape(-1), m3.reshape(-1))
    tc_partials = _tc_call(p3, g3, m3)
    total = jnp.sum(sc_partials) + jnp.sum(tc_partials)
    return total / (B * N * D)
